# baseline (speedup 1.0000x reference)
import functools

import jax
import jax.numpy as jnp
from jax import lax
from jax.experimental import pallas as pl
from jax.experimental.pallas import tpu as pltpu
from jax.experimental.pallas import tpu_sc as plsc

DIM = 2048
NC = 2
NS = 16
CH = 16
TN = 256


def _gather_rows(table, ids_flat):
    ntok = ids_flat.shape[0]
    n_work = NC * NS
    b_per_w = ntok // n_work
    mesh = plsc.VectorSubcoreMesh(core_axis_name="c", subcore_axis_name="s")

    @functools.partial(
        pl.kernel,
        mesh=mesh,
        out_type=jax.ShapeDtypeStruct((ntok, DIM), table.dtype),
        scratch_types=[
            pltpu.VMEM((b_per_w,), jnp.int32),
            pltpu.VMEM((CH, DIM), table.dtype),
            pltpu.SemaphoreType.DMA,
        ],
    )
    def gather_kernel(table_hbm, idx_hbm, out_hbm, idx_v, rows_v, sem):
        wid = lax.axis_index("s") * NC + lax.axis_index("c")
        base = wid * b_per_w
        pltpu.sync_copy(idx_hbm.at[pl.ds(base, b_per_w)], idx_v)

        @pl.loop(0, b_per_w // CH)
        def _(j):
            off = j * CH
            pltpu.async_copy(
                table_hbm.at[idx_v.at[pl.ds(off, CH)]], rows_v, sem
            ).wait()
            pltpu.sync_copy(rows_v, out_hbm.at[pl.ds(base + off, CH)])

    return gather_kernel(table, ids_flat)


def _project_half(h, W, b2d, half_idx, prev):
    ntok_half, vocab = h.shape[0], W.shape[0]
    total = 2 * ntok_half

    def mm_kernel(*refs):
        h_ref, w_ref, b_ref, o_ref = refs[-4:]
        o_ref[...] = jax.lax.dot_general(
            h_ref[...], w_ref[...],
            (((1,), (1,)), ((), ())),
            preferred_element_type=jnp.float32,
        ) + b_ref[...]

    data_specs = [
        pl.BlockSpec((ntok_half, DIM), lambda i: (0, 0)),
        pl.BlockSpec((TN, DIM), lambda i: (i, 0)),
        pl.BlockSpec((1, TN), lambda i: (0, i)),
    ]
    out_spec = pl.BlockSpec(
        (ntok_half, TN), lambda i, _h=half_idx: (_h, i)
    )
    out_type = jax.ShapeDtypeStruct((total, vocab), jnp.float32)
    params = pltpu.CompilerParams(dimension_semantics=("parallel",))
    if prev is None:
        return pl.pallas_call(
            mm_kernel,
            grid=(vocab // TN,),
            in_specs=data_specs,
            out_specs=out_spec,
            out_shape=out_type,
            compiler_params=params,
        )(h, W, b2d)
    return pl.pallas_call(
        mm_kernel,
        grid=(vocab // TN,),
        in_specs=[pl.BlockSpec(memory_space=pl.ANY)] + data_specs,
        out_specs=out_spec,
        out_shape=out_type,
        input_output_aliases={0: 0},
        compiler_params=params,
    )(prev, h, W, b2d)


def kernel(ids, emb_table, W, b):
    batch, seq = ids.shape
    ntok = batch * seq
    ids_flat = ids.reshape(ntok).astype(jnp.int32)
    b2d = b.reshape(1, -1)
    half = ntok // 2
    h0 = _gather_rows(emb_table, ids_flat[:half])
    h1 = _gather_rows(emb_table, ids_flat[half:])
    l0 = _project_half(h0, W, b2d, 0, None)
    logits = _project_half(h1, W, b2d, 1, l0)
    return logits.reshape(batch, seq, W.shape[0])


# speedup vs baseline: 1.0465x; 1.0465x over previous
import functools

import jax
import jax.numpy as jnp
from jax import lax
from jax.experimental import pallas as pl
from jax.experimental.pallas import tpu as pltpu
from jax.experimental.pallas import tpu_sc as plsc

DIM = 2048
NC = 2
NS = 16
CH = 16
TN = 256


def _gather_rows(table, ids_flat):
    ntok = ids_flat.shape[0]
    n_work = NC * NS
    b_per_w = ntok // n_work
    mesh = plsc.VectorSubcoreMesh(core_axis_name="c", subcore_axis_name="s")

    @functools.partial(
        pl.kernel,
        mesh=mesh,
        out_type=jax.ShapeDtypeStruct((ntok, DIM), table.dtype),
        scratch_types=[
            pltpu.VMEM((b_per_w,), jnp.int32),
            pltpu.VMEM((CH, DIM), table.dtype),
            pltpu.SemaphoreType.DMA,
        ],
    )
    def gather_kernel(table_hbm, idx_hbm, out_hbm, idx_v, rows_v, sem):
        wid = lax.axis_index("s") * NC + lax.axis_index("c")
        base = wid * b_per_w
        pltpu.sync_copy(idx_hbm.at[pl.ds(base, b_per_w)], idx_v)

        @pl.loop(0, b_per_w // CH)
        def _(j):
            off = j * CH
            pltpu.async_copy(
                table_hbm.at[idx_v.at[pl.ds(off, CH)]], rows_v, sem
            ).wait()
            pltpu.sync_copy(rows_v, out_hbm.at[pl.ds(base + off, CH)])

    return gather_kernel(table, ids_flat)


def _project(h, W, b2d):
    ntok, vocab = h.shape[0], W.shape[0]

    chunk = 512

    def mm_kernel(h_hbm, w_ref, b_ref, o_ref, hbf_ref, t0, t1, sem0, sem1):
        @pl.when(pl.program_id(0) == 0)
        def _():
            tmps, sems = (t0, t1), (sem0, sem1)
            n_chunks = ntok // chunk
            copies = []
            for c in range(n_chunks):
                cp = pltpu.make_async_copy(
                    h_hbm.at[pl.ds(c * chunk, chunk), :], tmps[c % 2], sems[c % 2]
                )
                cp.start()
                copies.append(cp)
                if c >= 1:
                    copies[c - 1].wait()
                    hbf_ref[pl.ds((c - 1) * chunk, chunk), :] = (
                        tmps[(c - 1) % 2][...].astype(jnp.bfloat16)
                    )
            copies[-1].wait()
            hbf_ref[pl.ds((n_chunks - 1) * chunk, chunk), :] = (
                tmps[(n_chunks - 1) % 2][...].astype(jnp.bfloat16)
            )

        o_ref[...] = jax.lax.dot_general(
            hbf_ref[...], w_ref[...].astype(jnp.bfloat16),
            (((1,), (1,)), ((), ())),
            preferred_element_type=jnp.float32,
        ) + b_ref[...]

    return pl.pallas_call(
        mm_kernel,
        grid=(vocab // TN,),
        in_specs=[
            pl.BlockSpec(memory_space=pl.ANY),
            pl.BlockSpec((TN, DIM), lambda i: (i, 0)),
            pl.BlockSpec((1, TN), lambda i: (0, i)),
        ],
        out_specs=pl.BlockSpec((ntok, TN), lambda i: (0, i)),
        out_shape=jax.ShapeDtypeStruct((ntok, vocab), jnp.float32),
        scratch_shapes=[
            pltpu.VMEM((ntok, DIM), jnp.bfloat16),
            pltpu.VMEM((chunk, DIM), jnp.float32),
            pltpu.VMEM((chunk, DIM), jnp.float32),
            pltpu.SemaphoreType.DMA,
            pltpu.SemaphoreType.DMA,
        ],
        compiler_params=pltpu.CompilerParams(
            dimension_semantics=("arbitrary",),
        ),
    )(h, W, b2d)


def kernel(ids, emb_table, W, b):
    batch, seq = ids.shape
    ntok = batch * seq
    ids_flat = ids.reshape(ntok).astype(jnp.int32)
    b2d = b.reshape(1, -1)
    h = _gather_rows(emb_table, ids_flat)
    logits = _project(h, W, b2d)
    return logits.reshape(batch, seq, W.shape[0])
